# Initial kernel scaffold; baseline (speedup 1.0000x reference)
import jax
import jax.numpy as jnp
from jax.experimental import pallas as pl


def kernel(x, train_pos_edge_index, edge_weight, W1, att_src1, att_dst1, b1, W2, att_src2, att_dst2, b2):
    raise NotImplementedError("write your pallas kernel here")


# speedup vs baseline: 20.3240x; 20.3240x over previous
import functools

import jax
import jax.numpy as jnp
from jax import lax
from jax.experimental import pallas as pl
from jax.experimental.pallas import tpu as pltpu
from jax.experimental.pallas import tpu_sc as plsc

N = 10000
D_IN = 128
H1 = 128
H2 = 16
NEG = 0.2

NT = 16
NC = 2
NP = 10240
DUMMY = N
B = 128
NBLK = 81
EPP = NC * NT * NBLK * B
RPT = NP // NT


def _tc1_body(x_ref, w_ref, asw_ref, adw_ref, h_ref, as_ref, ad_ref):
    h = jnp.dot(x_ref[...], w_ref[...], preferred_element_type=jnp.float32)
    h_ref[...] = h
    as_ref[...] = jnp.sum(h * asw_ref[...], axis=1, keepdims=True)
    ad_ref[...] = jnp.sum(h * adw_ref[...], axis=1, keepdims=True)


def _tc1(x, W1, asw, adw):
    blk = 1000
    return pl.pallas_call(
        _tc1_body,
        grid=(N // blk,),
        in_specs=[
            pl.BlockSpec((blk, D_IN), lambda i: (i, 0)),
            pl.BlockSpec((D_IN, H1), lambda i: (0, 0)),
            pl.BlockSpec((1, H1), lambda i: (0, 0)),
            pl.BlockSpec((1, H1), lambda i: (0, 0)),
        ],
        out_specs=[
            pl.BlockSpec((blk, H1), lambda i: (i, 0)),
            pl.BlockSpec((blk, 1), lambda i: (i, 0)),
            pl.BlockSpec((blk, 1), lambda i: (i, 0)),
        ],
        out_shape=[
            jax.ShapeDtypeStruct((N, H1), jnp.float32),
            jax.ShapeDtypeStruct((N, 1), jnp.float32),
            jax.ShapeDtypeStruct((N, 1), jnp.float32),
        ],
    )(x, W1, asw, adw)


def _tc2_body(m0, m1, dt, b1r, w2, asw, adw, h1p_ref, as_ref, ad_ref):
    den = jnp.sum(dt[...], axis=1, keepdims=True) + 1e-16
    h1 = jnp.maximum((m0[...] + m1[...]) / den + b1r[...], 0.0)
    h1p_ref[...] = h1
    h2 = jnp.dot(h1, w2[...], preferred_element_type=jnp.float32)
    as_ref[...] = jnp.sum(h2 * asw[...], axis=1, keepdims=True)
    ad_ref[...] = jnp.sum(h2 * adw[...], axis=1, keepdims=True)


def _tc2(m0, m1, dt, b1r, W2, asw, adw):
    blk = 1024
    return pl.pallas_call(
        _tc2_body,
        grid=(NP // blk,),
        in_specs=[
            pl.BlockSpec((blk, H1), lambda i: (i, 0)),
            pl.BlockSpec((blk, H1), lambda i: (i, 0)),
            pl.BlockSpec((blk, NC * NT), lambda i: (i, 0)),
            pl.BlockSpec((1, H1), lambda i: (0, 0)),
            pl.BlockSpec((H1, H2), lambda i: (0, 0)),
            pl.BlockSpec((1, H2), lambda i: (0, 0)),
            pl.BlockSpec((1, H2), lambda i: (0, 0)),
        ],
        out_specs=[
            pl.BlockSpec((blk, H1), lambda i: (i, 0)),
            pl.BlockSpec((blk, 1), lambda i: (i, 0)),
            pl.BlockSpec((blk, 1), lambda i: (i, 0)),
        ],
        out_shape=[
            jax.ShapeDtypeStruct((NP, H1), jnp.float32),
            jax.ShapeDtypeStruct((NP, 1), jnp.float32),
            jax.ShapeDtypeStruct((NP, 1), jnp.float32),
        ],
    )(m0, m1, dt, b1r, W2, asw, adw)


def _tc3_body(m0, m1, dt, w2, b2r, out_ref):
    den = jnp.sum(dt[...], axis=1, keepdims=True) + 1e-16
    agg = (m0[...] + m1[...]) / den
    out_ref[...] = (
        jnp.dot(agg, w2[...], preferred_element_type=jnp.float32) + b2r[...])


def _tc3(m0, m1, dt, W2, b2r):
    blk = 1024
    return pl.pallas_call(
        _tc3_body,
        grid=(NP // blk,),
        in_specs=[
            pl.BlockSpec((blk, H1), lambda i: (i, 0)),
            pl.BlockSpec((blk, H1), lambda i: (i, 0)),
            pl.BlockSpec((blk, NC * NT), lambda i: (i, 0)),
            pl.BlockSpec((H1, H2), lambda i: (0, 0)),
            pl.BlockSpec((1, H2), lambda i: (0, 0)),
        ],
        out_specs=pl.BlockSpec((blk, H2), lambda i: (i, 0)),
        out_shape=jax.ShapeDtypeStruct((NP, H2), jnp.float32),
    )(m0, m1, dt, W2, b2r)


def _make_sc_layer(C, SB):
    R = B // SB
    mesh = plsc.VectorSubcoreMesh(core_axis_name="c", subcore_axis_name="s",
                                  num_cores=NC, num_subcores=NT)

    @functools.partial(
        pl.kernel,
        out_type=(
            jax.ShapeDtypeStruct((NC, NP, C), jnp.float32),
            jax.ShapeDtypeStruct((NC * NT * NP,), jnp.float32),
        ),
        mesh=mesh,
        compiler_params=pltpu.CompilerParams(needs_layout_passes=False),
        scratch_types=(
            [pltpu.VMEM((NP,), jnp.float32),
             pltpu.VMEM((NP,), jnp.float32),
             pltpu.VMEM((NP,), jnp.float32)]
            + [pltpu.VMEM((R,), jnp.int32) for _ in range(2 * SB)]
            + [pltpu.VMEM((R,), jnp.float32) for _ in range(SB)]
            + [pltpu.VMEM((R, C), jnp.float32),
               pltpu.VMEM_SHARED((NP, C), jnp.float32),
               pltpu.SemaphoreType.DMA]
        ),
    )
    def sc_layer(h_hbm, asrc_hbm, adst_hbm, src_hbm, dst_hbm,
                 macc_hbm, den_hbm, asrc_v, adst_v, den_v, *rest):
        srcbs = rest[0:SB]
        dstbs = rest[SB:2 * SB]
        exbs = rest[2 * SB:3 * SB]
        hrows, acc_sh, sem = rest[3 * SB:]
        cid = lax.axis_index("c")
        sid = lax.axis_index("s")
        wid = sid * NC + cid
        z = jnp.zeros((16,), jnp.float32)

        def zrow(r, _):
            for k in range(C // 16):
                hrows[r, pl.ds(k * 16, 16)] = z
            return 0
        lax.fori_loop(0, R, zrow, 0)

        def zden(i, _):
            den_v[pl.ds(i * 16, 16)] = z
            return 0
        lax.fori_loop(0, NP // 16, zden, 0)
        base = sid * RPT
        for b in range(RPT // R):
            pltpu.sync_copy(hrows, acc_sh.at[pl.ds(base + b * R, R)])

        pltpu.sync_copy(asrc_hbm, asrc_v)
        pltpu.sync_copy(adst_hbm, adst_v)
        plsc.subcore_barrier()

        def blk(j, _):
            for s in range(SB):
                pltpu.sync_copy(src_hbm.at[wid, j, s], srcbs[s])
                pltpu.sync_copy(dst_hbm.at[wid, j, s], dstbs[s])
            for s in range(SB):
                for i in range(R // 16):
                    sv = srcbs[s][pl.ds(i * 16, 16)]
                    dv = dstbs[s][pl.ds(i * 16, 16)]
                    a = (plsc.load_gather(asrc_v, [sv])
                         + plsc.load_gather(adst_v, [dv]))
                    a = jnp.where(a >= 0, a, a * NEG)
                    ex = jnp.exp(a)
                    exbs[s][pl.ds(i * 16, 16)] = ex
                    plsc.addupdate_scatter(den_v, [dv], ex)
            for s in range(SB):
                pltpu.async_copy(h_hbm.at[srcbs[s]], hrows, sem).wait()

                def med(i, _, s=s):
                    ev = exbs[s][pl.ds(i * 16, 16)]
                    for l in range(16):
                        bex = jnp.full((16,), ev[l], jnp.float32)
                        jj = i * 16 + l
                        for k in range(C // 16):
                            hrows[jj, pl.ds(k * 16, 16)] = (
                                hrows[jj, pl.ds(k * 16, 16)] * bex)
                    return 0
                lax.fori_loop(0, R // 16, med, 0)
                pltpu.sync_copy(hrows, acc_sh.at[dstbs[s]], add=True)
            return 0
        lax.fori_loop(0, NBLK, blk, 0)

        plsc.subcore_barrier()
        pltpu.sync_copy(acc_sh.at[pl.ds(base, RPT)],
                        macc_hbm.at[cid, pl.ds(base, RPT)])
        pltpu.sync_copy(den_v, den_hbm.at[pl.ds((cid * NT + sid) * NP, NP)])

    return sc_layer


_sc128 = _make_sc_layer(H1, 2)


def kernel(x, train_pos_edge_index, edge_weight, W1, att_src1, att_dst1, b1,
           W2, att_src2, att_dst2, b2):
    del edge_weight
    ei = train_pos_edge_index.astype(jnp.int32)
    loop = jnp.arange(N, dtype=jnp.int32)
    pad = EPP - (ei.shape[1] + N)
    src = jnp.concatenate([ei[0], loop, jnp.zeros((pad,), jnp.int32)])
    dst = jnp.concatenate([ei[1], loop, jnp.full((pad,), DUMMY, jnp.int32)])
    src_q2 = src.reshape(NC * NT, NBLK, 2, B // 2)
    dst_q2 = dst.reshape(NC * NT, NBLK, 2, B // 2)

    h1, a_s, a_d = _tc1(x, W1, att_src1.reshape(1, H1), att_dst1.reshape(1, H1))
    asp = jnp.pad(a_s.reshape(-1), (0, NP - N))
    adp = jnp.pad(a_d.reshape(-1), (0, NP - N))
    macc1, den1 = _sc128(h1, asp, adp, src_q2, dst_q2)
    den1t = den1.reshape(NC * NT, NP).T
    h1p, as2, ad2 = _tc2(macc1[0], macc1[1], den1t,
                         b1.reshape(1, H1), W2,
                         att_src2.reshape(1, H2), att_dst2.reshape(1, H2))
    macc2, den2 = _sc128(h1p, as2.reshape(-1), ad2.reshape(-1), src_q2, dst_q2)
    den2t = den2.reshape(NC * NT, NP).T
    out = _tc3(macc2[0], macc2[1], den2t, W2, b2.reshape(1, H2))
    return out[:N]


# speedup vs baseline: 25.5205x; 1.2557x over previous
import functools

import jax
import jax.numpy as jnp
from jax import lax
from jax.experimental import pallas as pl
from jax.experimental.pallas import tpu as pltpu
from jax.experimental.pallas import tpu_sc as plsc

N = 10000
D_IN = 128
H1 = 128
H2 = 16
NEG = 0.2

NT = 16
NC = 2
NP = 10112
DUMMY = N
R = 48
PAIRS = 108
EPP = NC * NT * PAIRS * 2 * R
RPT = NP // NT


def _tc1_body(x_ref, w_ref, asw_ref, adw_ref, h_ref, as_ref, ad_ref):
    h = jnp.dot(x_ref[...], w_ref[...], preferred_element_type=jnp.float32)
    h_ref[...] = h
    as_ref[...] = jnp.sum(h * asw_ref[...], axis=1, keepdims=True)
    ad_ref[...] = jnp.sum(h * adw_ref[...], axis=1, keepdims=True)


def _tc1(x, W1, asw, adw):
    blk = 1000
    return pl.pallas_call(
        _tc1_body,
        grid=(N // blk,),
        in_specs=[
            pl.BlockSpec((blk, D_IN), lambda i: (i, 0)),
            pl.BlockSpec((D_IN, H1), lambda i: (0, 0)),
            pl.BlockSpec((1, H1), lambda i: (0, 0)),
            pl.BlockSpec((1, H1), lambda i: (0, 0)),
        ],
        out_specs=[
            pl.BlockSpec((blk, H1), lambda i: (i, 0)),
            pl.BlockSpec((blk, 1), lambda i: (i, 0)),
            pl.BlockSpec((blk, 1), lambda i: (i, 0)),
        ],
        out_shape=[
            jax.ShapeDtypeStruct((N, H1), jnp.float32),
            jax.ShapeDtypeStruct((N, 1), jnp.float32),
            jax.ShapeDtypeStruct((N, 1), jnp.float32),
        ],
    )(x, W1, asw, adw)


def _tc2_body(m0, m1, dt, b1r, w2, asw, adw, h1p_ref, as_ref, ad_ref):
    den = jnp.sum(dt[...], axis=1, keepdims=True) + 1e-16
    h1 = jnp.maximum((m0[...] + m1[...]) / den + b1r[...], 0.0)
    h1p_ref[...] = h1
    h2 = jnp.dot(h1, w2[...], preferred_element_type=jnp.float32)
    as_ref[...] = jnp.sum(h2 * asw[...], axis=1, keepdims=True)
    ad_ref[...] = jnp.sum(h2 * adw[...], axis=1, keepdims=True)


def _tc2(m0, m1, dt, b1r, W2, asw, adw):
    return pl.pallas_call(
        _tc2_body,
        grid=(1,),
        in_specs=[
            pl.BlockSpec((NP, H1), lambda i: (0, 0)),
            pl.BlockSpec((NP, H1), lambda i: (0, 0)),
            pl.BlockSpec((NP, NC * NT), lambda i: (0, 0)),
            pl.BlockSpec((1, H1), lambda i: (0, 0)),
            pl.BlockSpec((H1, H2), lambda i: (0, 0)),
            pl.BlockSpec((1, H2), lambda i: (0, 0)),
            pl.BlockSpec((1, H2), lambda i: (0, 0)),
        ],
        out_specs=[
            pl.BlockSpec((NP, H1), lambda i: (0, 0)),
            pl.BlockSpec((NP, 1), lambda i: (0, 0)),
            pl.BlockSpec((NP, 1), lambda i: (0, 0)),
        ],
        out_shape=[
            jax.ShapeDtypeStruct((NP, H1), jnp.float32),
            jax.ShapeDtypeStruct((NP, 1), jnp.float32),
            jax.ShapeDtypeStruct((NP, 1), jnp.float32),
        ],
    )(m0, m1, dt, b1r, W2, asw, adw)


def _tc3_body(m0, m1, dt, w2, b2r, out_ref):
    den = jnp.sum(dt[...], axis=1, keepdims=True) + 1e-16
    agg = (m0[...] + m1[...]) / den
    out_ref[...] = (
        jnp.dot(agg, w2[...], preferred_element_type=jnp.float32) + b2r[...])


def _tc3(m0, m1, dt, W2, b2r):
    return pl.pallas_call(
        _tc3_body,
        grid=(1,),
        in_specs=[
            pl.BlockSpec((NP, H1), lambda i: (0, 0)),
            pl.BlockSpec((NP, H1), lambda i: (0, 0)),
            pl.BlockSpec((NP, NC * NT), lambda i: (0, 0)),
            pl.BlockSpec((H1, H2), lambda i: (0, 0)),
            pl.BlockSpec((1, H2), lambda i: (0, 0)),
        ],
        out_specs=pl.BlockSpec((NP, H2), lambda i: (0, 0)),
        out_shape=jax.ShapeDtypeStruct((NP, H2), jnp.float32),
    )(m0, m1, dt, W2, b2r)


def _make_sc_layer(C):
    mesh = plsc.VectorSubcoreMesh(core_axis_name="c", subcore_axis_name="s",
                                  num_cores=NC, num_subcores=NT)

    @functools.partial(
        pl.kernel,
        out_type=(
            jax.ShapeDtypeStruct((NC, NP, C), jnp.float32),
            jax.ShapeDtypeStruct((NC * NT * NP,), jnp.float32),
        ),
        mesh=mesh,
        compiler_params=pltpu.CompilerParams(needs_layout_passes=False),
        scratch_types=[
            pltpu.VMEM((NP,), jnp.float32),
            pltpu.VMEM((NP,), jnp.float32),
            pltpu.VMEM((NP,), jnp.float32),
            pltpu.VMEM((R,), jnp.int32),
            pltpu.VMEM((R,), jnp.int32),
            pltpu.VMEM((R,), jnp.int32),
            pltpu.VMEM((R,), jnp.int32),
            pltpu.VMEM((R,), jnp.float32),
            pltpu.VMEM((R,), jnp.float32),
            pltpu.VMEM((R, C), jnp.float32),
            pltpu.VMEM((R, C), jnp.float32),
            pltpu.VMEM_SHARED((NP, C), jnp.float32),
            pltpu.SemaphoreType.DMA,
            pltpu.SemaphoreType.DMA,
            pltpu.SemaphoreType.DMA,
            pltpu.SemaphoreType.DMA,
        ],
    )
    def sc_layer(h_hbm, asrc_hbm, adst_hbm, src_hbm, dst_hbm,
                 macc_hbm, den_hbm,
                 asrc_v, adst_v, den_v, srcb_a, srcb_b, dstb_a, dstb_b,
                 exb_a, exb_b, hb_a, hb_b, acc_sh,
                 sem_g, sem_i, sem_sa, sem_sb):
        cid = lax.axis_index("c")
        sid = lax.axis_index("s")
        wid = sid * NC + cid
        z = jnp.zeros((16,), jnp.float32)

        def zrow(r, _):
            for k in range(C // 16):
                hb_a[r, pl.ds(k * 16, 16)] = z
            return 0
        lax.fori_loop(0, R, zrow, 0)

        def zden(i, _):
            den_v[pl.ds(i * 16, 16)] = z
            return 0
        lax.fori_loop(0, NP // 16, zden, 0)
        base = sid * RPT
        for b in range(RPT // R):
            pltpu.sync_copy(hb_a, acc_sh.at[pl.ds(base + b * R, R)])
        rem = RPT % R
        if rem:
            pltpu.sync_copy(hb_a.at[pl.ds(0, rem)],
                            acc_sh.at[pl.ds(base + (RPT // R) * R, rem)])

        pltpu.sync_copy(asrc_hbm, asrc_v)
        pltpu.sync_copy(adst_hbm, adst_v)
        plsc.subcore_barrier()

        def logits(srcb, dstb, exb):
            for i in range(R // 16):
                sv = srcb[pl.ds(i * 16, 16)]
                dv = dstb[pl.ds(i * 16, 16)]
                a = (plsc.load_gather(asrc_v, [sv])
                     + plsc.load_gather(adst_v, [dv]))
                a = jnp.where(a >= 0, a, a * NEG)
                ex = jnp.exp(a)
                exb[pl.ds(i * 16, 16)] = ex
                plsc.addupdate_scatter(den_v, [dv], ex)

        def med(hb, exb):
            def body(i, _):
                ev = exb[pl.ds(i * 16, 16)]
                for l in range(16):
                    bex = jnp.full((16,), ev[l], jnp.float32)
                    jj = i * 16 + l
                    for k in range(C // 16):
                        hb[jj, pl.ds(k * 16, 16)] = (
                            hb[jj, pl.ds(k * 16, 16)] * bex)
                return 0
            lax.fori_loop(0, R // 16, body, 0)

        for d in [pltpu.async_copy(src_hbm.at[wid, 0, 0], srcb_a, sem_i),
                  pltpu.async_copy(dst_hbm.at[wid, 0, 0], dstb_a, sem_i),
                  pltpu.async_copy(src_hbm.at[wid, 0, 1], srcb_b, sem_i),
                  pltpu.async_copy(dst_hbm.at[wid, 0, 1], dstb_b, sem_i)]:
            d.wait()

        def pair(k, _):
            ga = pltpu.async_copy(h_hbm.at[srcb_a], hb_a, sem_g)
            logits(srcb_a, dstb_a, exb_a)
            ga.wait()
            gb = pltpu.async_copy(h_hbm.at[srcb_b], hb_b, sem_g)
            med(hb_a, exb_a)
            sa = pltpu.async_copy(hb_a, acc_sh.at[dstb_a], sem_sa, add=True)
            logits(srcb_b, dstb_b, exb_b)
            gb.wait()
            med(hb_b, exb_b)
            sb = pltpu.async_copy(hb_b, acc_sh.at[dstb_b], sem_sb, add=True)
            sa.wait()
            sb.wait()
            kn = jnp.minimum(k + 1, PAIRS - 1)
            for d in [pltpu.async_copy(src_hbm.at[wid, kn, 0], srcb_a, sem_i),
                      pltpu.async_copy(dst_hbm.at[wid, kn, 0], dstb_a, sem_i),
                      pltpu.async_copy(src_hbm.at[wid, kn, 1], srcb_b, sem_i),
                      pltpu.async_copy(dst_hbm.at[wid, kn, 1], dstb_b, sem_i)]:
                d.wait()
            return 0
        lax.fori_loop(0, PAIRS, pair, 0)

        plsc.subcore_barrier()
        pltpu.sync_copy(acc_sh.at[pl.ds(base, RPT)],
                        macc_hbm.at[cid, pl.ds(base, RPT)])
        pltpu.sync_copy(den_v, den_hbm.at[pl.ds((cid * NT + sid) * NP, NP)])

    return sc_layer


_sc128 = _make_sc_layer(H1)


def kernel(x, train_pos_edge_index, edge_weight, W1, att_src1, att_dst1, b1,
           W2, att_src2, att_dst2, b2):
    del edge_weight
    ei = train_pos_edge_index.astype(jnp.int32)
    loop = jnp.arange(N, dtype=jnp.int32)
    pad = EPP - (ei.shape[1] + N)
    src = jnp.concatenate([ei[0], loop, jnp.zeros((pad,), jnp.int32)])
    dst = jnp.concatenate([ei[1], loop, jnp.full((pad,), DUMMY, jnp.int32)])
    src_q = src.reshape(NC * NT, PAIRS, 2, R)
    dst_q = dst.reshape(NC * NT, PAIRS, 2, R)

    h1, a_s, a_d = _tc1(x, W1, att_src1.reshape(1, H1), att_dst1.reshape(1, H1))
    asp = jnp.pad(a_s.reshape(-1), (0, NP - N))
    adp = jnp.pad(a_d.reshape(-1), (0, NP - N))
    macc1, den1 = _sc128(h1, asp, adp, src_q, dst_q)
    den1t = den1.reshape(NC * NT, NP).T
    h1p, as2, ad2 = _tc2(macc1[0], macc1[1], den1t,
                         b1.reshape(1, H1), W2,
                         att_src2.reshape(1, H2), att_dst2.reshape(1, H2))
    macc2, den2 = _sc128(h1p, as2.reshape(-1), ad2.reshape(-1), src_q, dst_q)
    den2t = den2.reshape(NC * NT, NP).T
    out = _tc3(macc2[0], macc2[1], den2t, W2, b2.reshape(1, H2))
    return out[:N]


# speedup vs baseline: 25.8277x; 1.0120x over previous
import functools

import jax
import jax.numpy as jnp
from jax import lax
from jax.experimental import pallas as pl
from jax.experimental.pallas import tpu as pltpu
from jax.experimental.pallas import tpu_sc as plsc

N = 10000
D_IN = 128
H1 = 128
H2 = 16
NEG = 0.2

NT = 16
NC = 2
NP = 10112
DUMMY = N
R = 48
PAIRS = 108
EPP = NC * NT * PAIRS * 2 * R
RPT = NP // NT


def _tc1_body(x_ref, w_ref, asw_ref, adw_ref, h_ref, as_ref, ad_ref):
    h = jnp.dot(x_ref[...], w_ref[...], preferred_element_type=jnp.float32)
    h_ref[...] = h
    as_ref[...] = jnp.sum(h * asw_ref[...], axis=1, keepdims=True)
    ad_ref[...] = jnp.sum(h * adw_ref[...], axis=1, keepdims=True)


def _tc1(x, W1, asw, adw):
    blk = 1000
    return pl.pallas_call(
        _tc1_body,
        grid=(N // blk,),
        in_specs=[
            pl.BlockSpec((blk, D_IN), lambda i: (i, 0)),
            pl.BlockSpec((D_IN, H1), lambda i: (0, 0)),
            pl.BlockSpec((1, H1), lambda i: (0, 0)),
            pl.BlockSpec((1, H1), lambda i: (0, 0)),
        ],
        out_specs=[
            pl.BlockSpec((blk, H1), lambda i: (i, 0)),
            pl.BlockSpec((blk, 1), lambda i: (i, 0)),
            pl.BlockSpec((blk, 1), lambda i: (i, 0)),
        ],
        out_shape=[
            jax.ShapeDtypeStruct((N, H1), jnp.float32),
            jax.ShapeDtypeStruct((N, 1), jnp.float32),
            jax.ShapeDtypeStruct((N, 1), jnp.float32),
        ],
    )(x, W1, asw, adw)


def _tc2_body(m0, m1, dt, b1r, w2, asw, adw, h1p_ref, as_ref, ad_ref):
    den = jnp.sum(dt[...], axis=1, keepdims=True) + 1e-16
    h1 = jnp.maximum((m0[...] + m1[...]) / den + b1r[...], 0.0)
    h1p_ref[...] = h1
    h2 = jnp.dot(h1, w2[...], preferred_element_type=jnp.float32)
    as_ref[...] = jnp.sum(h2 * asw[...], axis=1, keepdims=True)
    ad_ref[...] = jnp.sum(h2 * adw[...], axis=1, keepdims=True)


def _tc2(m0, m1, dt, b1r, W2, asw, adw):
    return pl.pallas_call(
        _tc2_body,
        grid=(1,),
        in_specs=[
            pl.BlockSpec((NP, H1), lambda i: (0, 0)),
            pl.BlockSpec((NP, H1), lambda i: (0, 0)),
            pl.BlockSpec((NP, NC * NT), lambda i: (0, 0)),
            pl.BlockSpec((1, H1), lambda i: (0, 0)),
            pl.BlockSpec((H1, H2), lambda i: (0, 0)),
            pl.BlockSpec((1, H2), lambda i: (0, 0)),
            pl.BlockSpec((1, H2), lambda i: (0, 0)),
        ],
        out_specs=[
            pl.BlockSpec((NP, H1), lambda i: (0, 0)),
            pl.BlockSpec((NP, 1), lambda i: (0, 0)),
            pl.BlockSpec((NP, 1), lambda i: (0, 0)),
        ],
        out_shape=[
            jax.ShapeDtypeStruct((NP, H1), jnp.float32),
            jax.ShapeDtypeStruct((NP, 1), jnp.float32),
            jax.ShapeDtypeStruct((NP, 1), jnp.float32),
        ],
    )(m0, m1, dt, b1r, W2, asw, adw)


def _tc3_body(m0, m1, dt, w2, b2r, out_ref):
    den = jnp.sum(dt[...], axis=1, keepdims=True) + 1e-16
    agg = (m0[...] + m1[...]) / den
    out_ref[...] = (
        jnp.dot(agg, w2[...], preferred_element_type=jnp.float32) + b2r[...])


def _tc3(m0, m1, dt, W2, b2r):
    return pl.pallas_call(
        _tc3_body,
        grid=(1,),
        in_specs=[
            pl.BlockSpec((NP, H1), lambda i: (0, 0)),
            pl.BlockSpec((NP, H1), lambda i: (0, 0)),
            pl.BlockSpec((NP, NC * NT), lambda i: (0, 0)),
            pl.BlockSpec((H1, H2), lambda i: (0, 0)),
            pl.BlockSpec((1, H2), lambda i: (0, 0)),
        ],
        out_specs=pl.BlockSpec((NP, H2), lambda i: (0, 0)),
        out_shape=jax.ShapeDtypeStruct((NP, H2), jnp.float32),
    )(m0, m1, dt, W2, b2r)


def _make_sc_layer(C):
    mesh = plsc.VectorSubcoreMesh(core_axis_name="c", subcore_axis_name="s",
                                  num_cores=NC, num_subcores=NT)

    @functools.partial(
        pl.kernel,
        out_type=(
            jax.ShapeDtypeStruct((NC, NP, C), jnp.float32),
            jax.ShapeDtypeStruct((NC * NT * NP,), jnp.float32),
        ),
        mesh=mesh,
        compiler_params=pltpu.CompilerParams(needs_layout_passes=False),
        scratch_types=[
            pltpu.VMEM((NP,), jnp.float32),
            pltpu.VMEM((NP,), jnp.float32),
            pltpu.VMEM((NP,), jnp.float32),
            pltpu.VMEM((R,), jnp.int32),
            pltpu.VMEM((R,), jnp.int32),
            pltpu.VMEM((R,), jnp.int32),
            pltpu.VMEM((R,), jnp.int32),
            pltpu.VMEM((R,), jnp.float32),
            pltpu.VMEM((R,), jnp.float32),
            pltpu.VMEM((R, C), jnp.float32),
            pltpu.VMEM((R, C), jnp.float32),
            pltpu.VMEM_SHARED((NP, C), jnp.float32),
            pltpu.SemaphoreType.DMA,
            pltpu.SemaphoreType.DMA,
            pltpu.SemaphoreType.DMA,
            pltpu.SemaphoreType.DMA,
        ],
    )
    def sc_layer(h_hbm, asrc_hbm, adst_hbm, src_hbm, dst_hbm,
                 macc_hbm, den_hbm,
                 asrc_v, adst_v, den_v, srcb_a, srcb_b, dstb_a, dstb_b,
                 exb_a, exb_b, hb_a, hb_b, acc_sh,
                 sem_g, sem_i, sem_sa, sem_sb):
        cid = lax.axis_index("c")
        sid = lax.axis_index("s")
        wid = sid * NC + cid
        z = jnp.zeros((16,), jnp.float32)

        def zrow(r, _):
            for k in range(C // 16):
                hb_a[r, pl.ds(k * 16, 16)] = z
            return 0
        lax.fori_loop(0, R, zrow, 0)

        def zden(i, _):
            den_v[pl.ds(i * 16, 16)] = z
            return 0
        lax.fori_loop(0, NP // 16, zden, 0)
        base = sid * RPT
        for b in range(RPT // R):
            pltpu.sync_copy(hb_a, acc_sh.at[pl.ds(base + b * R, R)])
        rem = RPT % R
        if rem:
            pltpu.sync_copy(hb_a.at[pl.ds(0, rem)],
                            acc_sh.at[pl.ds(base + (RPT // R) * R, rem)])

        pltpu.sync_copy(asrc_hbm, asrc_v)
        pltpu.sync_copy(adst_hbm, adst_v)
        plsc.subcore_barrier()

        def logits(srcb, dstb, exb):
            for i in range(R // 16):
                sv = srcb[pl.ds(i * 16, 16)]
                dv = dstb[pl.ds(i * 16, 16)]
                a = (plsc.load_gather(asrc_v, [sv])
                     + plsc.load_gather(adst_v, [dv]))
                a = jnp.where(a >= 0, a, a * NEG)
                ex = jnp.exp(a)
                exb[pl.ds(i * 16, 16)] = ex
                plsc.addupdate_scatter(den_v, [dv], ex)

        def med(hb, exb):
            def body(i, _):
                ev = exb[pl.ds(i * 16, 16)]
                for l in range(16):
                    bex = jnp.full((16,), ev[l], jnp.float32)
                    jj = i * 16 + l
                    for k in range(C // 16):
                        hb[jj, pl.ds(k * 16, 16)] = (
                            hb[jj, pl.ds(k * 16, 16)] * bex)
                return 0
            lax.fori_loop(0, R // 16, body, 0)

        for d in [pltpu.async_copy(src_hbm.at[wid, 0, 0], srcb_a, sem_i),
                  pltpu.async_copy(dst_hbm.at[wid, 0, 0], dstb_a, sem_i),
                  pltpu.async_copy(src_hbm.at[wid, 0, 1], srcb_b, sem_i),
                  pltpu.async_copy(dst_hbm.at[wid, 0, 1], dstb_b, sem_i)]:
            d.wait()

        def pair(k, _):
            kn = jnp.minimum(k + 1, PAIRS - 1)
            ga = pltpu.async_copy(h_hbm.at[srcb_a], hb_a, sem_g)
            logits(srcb_a, dstb_a, exb_a)
            ga.wait()
            ia = pltpu.async_copy(src_hbm.at[wid, kn, 0], srcb_a, sem_i)
            gb = pltpu.async_copy(h_hbm.at[srcb_b], hb_b, sem_g)
            med(hb_a, exb_a)
            sa = pltpu.async_copy(hb_a, acc_sh.at[dstb_a], sem_sa, add=True)
            logits(srcb_b, dstb_b, exb_b)
            gb.wait()
            ib = pltpu.async_copy(src_hbm.at[wid, kn, 1], srcb_b, sem_i)
            med(hb_b, exb_b)
            sb = pltpu.async_copy(hb_b, acc_sh.at[dstb_b], sem_sb, add=True)
            sa.wait()
            ja = pltpu.async_copy(dst_hbm.at[wid, kn, 0], dstb_a, sem_i)
            sb.wait()
            jb = pltpu.async_copy(dst_hbm.at[wid, kn, 1], dstb_b, sem_i)
            ia.wait()
            ib.wait()
            ja.wait()
            jb.wait()
            return 0
        lax.fori_loop(0, PAIRS, pair, 0)

        plsc.subcore_barrier()
        pltpu.sync_copy(acc_sh.at[pl.ds(base, RPT)],
                        macc_hbm.at[cid, pl.ds(base, RPT)])
        pltpu.sync_copy(den_v, den_hbm.at[pl.ds((cid * NT + sid) * NP, NP)])

    return sc_layer


_sc128 = _make_sc_layer(H1)


def kernel(x, train_pos_edge_index, edge_weight, W1, att_src1, att_dst1, b1,
           W2, att_src2, att_dst2, b2):
    del edge_weight
    ei = train_pos_edge_index.astype(jnp.int32)
    loop = jnp.arange(N, dtype=jnp.int32)
    pad = EPP - (ei.shape[1] + N)
    src = jnp.concatenate([ei[0], loop, jnp.zeros((pad,), jnp.int32)])
    dst = jnp.concatenate([ei[1], loop, jnp.full((pad,), DUMMY, jnp.int32)])
    src_q = src.reshape(NC * NT, PAIRS, 2, R)
    dst_q = dst.reshape(NC * NT, PAIRS, 2, R)

    h1, a_s, a_d = _tc1(x, W1, att_src1.reshape(1, H1), att_dst1.reshape(1, H1))
    asp = jnp.pad(a_s.reshape(-1), (0, NP - N))
    adp = jnp.pad(a_d.reshape(-1), (0, NP - N))
    macc1, den1 = _sc128(h1, asp, adp, src_q, dst_q)
    den1t = den1.reshape(NC * NT, NP).T
    h1p, as2, ad2 = _tc2(macc1[0], macc1[1], den1t,
                         b1.reshape(1, H1), W2,
                         att_src2.reshape(1, H2), att_dst2.reshape(1, H2))
    macc2, den2 = _sc128(h1p, as2.reshape(-1), ad2.reshape(-1), src_q, dst_q)
    den2t = den2.reshape(NC * NT, NP).T
    out = _tc3(macc2[0], macc2[1], den2t, W2, b2.reshape(1, H2))
    return out[:N]


# speedup vs baseline: 29.2950x; 1.1343x over previous
import functools

import jax
import jax.numpy as jnp
from jax import lax
from jax.experimental import pallas as pl
from jax.experimental.pallas import tpu as pltpu
from jax.experimental.pallas import tpu_sc as plsc

N = 10000
D_IN = 128
H1 = 128
H2 = 16
NEG = 0.2

NT = 16
NC = 2
NP = 10112
DUMMY = N
R = 48
PAIRS = 108
EPP = NC * NT * PAIRS * 2 * R
RPT = NP // NT


def _tc1_body(x_ref, w_ref, asw_ref, adw_ref, h_ref, as_ref, ad_ref):
    h = jnp.dot(x_ref[...], w_ref[...], preferred_element_type=jnp.float32)
    h_ref[...] = h
    as_ref[...] = jnp.sum(h * asw_ref[...], axis=1, keepdims=True)
    ad_ref[...] = jnp.sum(h * adw_ref[...], axis=1, keepdims=True)


def _tc1(x, W1, asw, adw):
    blk = 1000
    return pl.pallas_call(
        _tc1_body,
        grid=(N // blk,),
        in_specs=[
            pl.BlockSpec((blk, D_IN), lambda i: (i, 0)),
            pl.BlockSpec((D_IN, H1), lambda i: (0, 0)),
            pl.BlockSpec((1, H1), lambda i: (0, 0)),
            pl.BlockSpec((1, H1), lambda i: (0, 0)),
        ],
        out_specs=[
            pl.BlockSpec((blk, H1), lambda i: (i, 0)),
            pl.BlockSpec((blk, 1), lambda i: (i, 0)),
            pl.BlockSpec((blk, 1), lambda i: (i, 0)),
        ],
        out_shape=[
            jax.ShapeDtypeStruct((N, H1), jnp.float32),
            jax.ShapeDtypeStruct((N, 1), jnp.float32),
            jax.ShapeDtypeStruct((N, 1), jnp.float32),
        ],
    )(x, W1, asw, adw)


def _tc2_body(m0, m1, dt, b1r, w2, asw, adw, h1p_ref, as_ref, ad_ref):
    den = jnp.sum(dt[...], axis=1, keepdims=True) + 1e-16
    h1 = jnp.maximum((m0[...] + m1[...]) / den + b1r[...], 0.0)
    h1p_ref[...] = h1
    h2 = jnp.dot(h1, w2[...], preferred_element_type=jnp.float32)
    as_ref[...] = jnp.sum(h2 * asw[...], axis=1, keepdims=True)
    ad_ref[...] = jnp.sum(h2 * adw[...], axis=1, keepdims=True)


def _tc2(m0, m1, dt, b1r, W2, asw, adw):
    return pl.pallas_call(
        _tc2_body,
        grid=(1,),
        in_specs=[
            pl.BlockSpec((NP, H1), lambda i: (0, 0)),
            pl.BlockSpec((NP, H1), lambda i: (0, 0)),
            pl.BlockSpec((NP, NC * NT), lambda i: (0, 0)),
            pl.BlockSpec((1, H1), lambda i: (0, 0)),
            pl.BlockSpec((H1, H2), lambda i: (0, 0)),
            pl.BlockSpec((1, H2), lambda i: (0, 0)),
            pl.BlockSpec((1, H2), lambda i: (0, 0)),
        ],
        out_specs=[
            pl.BlockSpec((NP, H1), lambda i: (0, 0)),
            pl.BlockSpec((NP, 1), lambda i: (0, 0)),
            pl.BlockSpec((NP, 1), lambda i: (0, 0)),
        ],
        out_shape=[
            jax.ShapeDtypeStruct((NP, H1), jnp.float32),
            jax.ShapeDtypeStruct((NP, 1), jnp.float32),
            jax.ShapeDtypeStruct((NP, 1), jnp.float32),
        ],
    )(m0, m1, dt, b1r, W2, asw, adw)


def _tc3_body(m0, m1, dt, w2, b2r, out_ref):
    den = jnp.sum(dt[...], axis=1, keepdims=True) + 1e-16
    agg = (m0[...] + m1[...]) / den
    out_ref[...] = (
        jnp.dot(agg, w2[...], preferred_element_type=jnp.float32) + b2r[...])


def _tc3(m0, m1, dt, W2, b2r):
    return pl.pallas_call(
        _tc3_body,
        grid=(1,),
        in_specs=[
            pl.BlockSpec((NP, H1), lambda i: (0, 0)),
            pl.BlockSpec((NP, H1), lambda i: (0, 0)),
            pl.BlockSpec((NP, NC * NT), lambda i: (0, 0)),
            pl.BlockSpec((H1, H2), lambda i: (0, 0)),
            pl.BlockSpec((1, H2), lambda i: (0, 0)),
        ],
        out_specs=pl.BlockSpec((NP, H2), lambda i: (0, 0)),
        out_shape=jax.ShapeDtypeStruct((NP, H2), jnp.float32),
    )(m0, m1, dt, W2, b2r)


def _make_sc_layer(C):
    mesh = plsc.VectorSubcoreMesh(core_axis_name="c", subcore_axis_name="s",
                                  num_cores=NC, num_subcores=NT)

    @functools.partial(
        pl.kernel,
        out_type=(
            jax.ShapeDtypeStruct((NC, NP, C), jnp.float32),
            jax.ShapeDtypeStruct((NC * NT * NP,), jnp.float32),
        ),
        mesh=mesh,
        compiler_params=pltpu.CompilerParams(needs_layout_passes=False),
        scratch_types=[
            pltpu.VMEM((NP,), jnp.float32),
            pltpu.VMEM((NP,), jnp.float32),
            pltpu.VMEM((NP,), jnp.float32),
            pltpu.VMEM((R,), jnp.int32),
            pltpu.VMEM((R,), jnp.int32),
            pltpu.VMEM((R,), jnp.int32),
            pltpu.VMEM((R,), jnp.int32),
            pltpu.VMEM((R,), jnp.int32),
            pltpu.VMEM((R,), jnp.int32),
            pltpu.VMEM((R,), jnp.float32),
            pltpu.VMEM((R,), jnp.float32),
            pltpu.VMEM((R, C), jnp.float32),
            pltpu.VMEM((R, C), jnp.float32),
            pltpu.VMEM_SHARED((NP, C), jnp.float32),
            pltpu.SemaphoreType.DMA,
            pltpu.SemaphoreType.DMA,
            pltpu.SemaphoreType.DMA,
            pltpu.SemaphoreType.DMA,
        ],
    )
    def sc_layer(h_hbm, asrc_hbm, adst_hbm, src_hbm, dst_hbm,
                 macc_hbm, den_hbm,
                 asrc_v, adst_v, den_v, srcb_a, srcb_b, dstb_a, dstb_b,
                 dsc_a, dsc_b, exb_a, exb_b, hb_a, hb_b, acc_sh,
                 sem_g, sem_i, sem_sa, sem_sb):
        cid = lax.axis_index("c")
        sid = lax.axis_index("s")
        wid = sid * NC + cid
        z = jnp.zeros((16,), jnp.float32)

        def zrow(r, _):
            for k in range(C // 16):
                hb_a[r, pl.ds(k * 16, 16)] = z
            return 0
        lax.fori_loop(0, R, zrow, 0)

        def zden(i, _):
            den_v[pl.ds(i * 16, 16)] = z
            return 0
        lax.fori_loop(0, NP // 16, zden, 0)
        base = sid * RPT
        for b in range(RPT // R):
            pltpu.sync_copy(hb_a, acc_sh.at[pl.ds(base + b * R, R)])
        rem = RPT % R
        if rem:
            pltpu.sync_copy(hb_a.at[pl.ds(0, rem)],
                            acc_sh.at[pl.ds(base + (RPT // R) * R, rem)])

        pltpu.sync_copy(asrc_hbm, asrc_v)
        pltpu.sync_copy(adst_hbm, adst_v)
        plsc.subcore_barrier()

        def logits(srcb, dstb, exb):
            for i in range(R // 16):
                sv = srcb[pl.ds(i * 16, 16)]
                dv = dstb[pl.ds(i * 16, 16)]
                a = (plsc.load_gather(asrc_v, [sv])
                     + plsc.load_gather(adst_v, [dv]))
                a = jnp.where(a >= 0, a, a * NEG)
                ex = jnp.exp(a)
                exb[pl.ds(i * 16, 16)] = ex
                plsc.addupdate_scatter(den_v, [dv], ex)

        def med(hb, exb):
            def body(i, _):
                ev = exb[pl.ds(i * 16, 16)]
                for l in range(16):
                    bex = jnp.full((16,), ev[l], jnp.float32)
                    jj = i * 16 + l
                    for k in range(C // 16):
                        hb[jj, pl.ds(k * 16, 16)] = (
                            hb[jj, pl.ds(k * 16, 16)] * bex)
                return 0
            lax.fori_loop(0, R // 16, body, 0)

        for d in [pltpu.async_copy(src_hbm.at[wid, 0, 0], srcb_a, sem_i),
                  pltpu.async_copy(dst_hbm.at[wid, 0, 0], dstb_a, sem_i),
                  pltpu.async_copy(src_hbm.at[wid, 0, 1], srcb_b, sem_i),
                  pltpu.async_copy(dst_hbm.at[wid, 0, 1], dstb_b, sem_i)]:
            d.wait()

        def icopy(srcb, dstb):
            for i in range(R // 16):
                dstb[pl.ds(i * 16, 16)] = srcb[pl.ds(i * 16, 16)]

        for i in range(R // 16):
            dsc_a[pl.ds(i * 16, 16)] = jnp.full((16,), DUMMY, jnp.int32)
            dsc_b[pl.ds(i * 16, 16)] = jnp.full((16,), DUMMY, jnp.int32)
        pltpu.async_copy(hb_a, acc_sh.at[dsc_a], sem_sa, add=True)
        pltpu.async_copy(hb_a, acc_sh.at[dsc_b], sem_sb, add=True)

        def pair(k, _):
            kn = jnp.minimum(k + 1, PAIRS - 1)
            pltpu.make_async_copy(h_hbm.at[pl.ds(0, R)], hb_a, sem_sa).wait()
            ga = pltpu.async_copy(h_hbm.at[srcb_a], hb_a, sem_g)
            logits(srcb_a, dstb_a, exb_a)
            icopy(dstb_a, dsc_a)
            ja = pltpu.async_copy(dst_hbm.at[wid, kn, 0], dstb_a, sem_i)
            ga.wait()
            ia = pltpu.async_copy(src_hbm.at[wid, kn, 0], srcb_a, sem_i)
            pltpu.make_async_copy(h_hbm.at[pl.ds(0, R)], hb_b, sem_sb).wait()
            gb = pltpu.async_copy(h_hbm.at[srcb_b], hb_b, sem_g)
            med(hb_a, exb_a)
            pltpu.async_copy(hb_a, acc_sh.at[dsc_a], sem_sa, add=True)
            logits(srcb_b, dstb_b, exb_b)
            icopy(dstb_b, dsc_b)
            jb = pltpu.async_copy(dst_hbm.at[wid, kn, 1], dstb_b, sem_i)
            gb.wait()
            ib = pltpu.async_copy(src_hbm.at[wid, kn, 1], srcb_b, sem_i)
            med(hb_b, exb_b)
            pltpu.async_copy(hb_b, acc_sh.at[dsc_b], sem_sb, add=True)
            ia.wait()
            ib.wait()
            ja.wait()
            jb.wait()
            return 0
        lax.fori_loop(0, PAIRS, pair, 0)
        pltpu.make_async_copy(h_hbm.at[pl.ds(0, R)], hb_a, sem_sa).wait()
        pltpu.make_async_copy(h_hbm.at[pl.ds(0, R)], hb_b, sem_sb).wait()

        plsc.subcore_barrier()
        pltpu.sync_copy(acc_sh.at[pl.ds(base, RPT)],
                        macc_hbm.at[cid, pl.ds(base, RPT)])
        pltpu.sync_copy(den_v, den_hbm.at[pl.ds((cid * NT + sid) * NP, NP)])

    return sc_layer


_sc128 = _make_sc_layer(H1)


def kernel(x, train_pos_edge_index, edge_weight, W1, att_src1, att_dst1, b1,
           W2, att_src2, att_dst2, b2):
    del edge_weight
    ei = train_pos_edge_index.astype(jnp.int32)
    loop = jnp.arange(N, dtype=jnp.int32)
    pad = EPP - (ei.shape[1] + N)
    src = jnp.concatenate([ei[0], loop, jnp.zeros((pad,), jnp.int32)])
    dst = jnp.concatenate([ei[1], loop, jnp.full((pad,), DUMMY, jnp.int32)])
    src_q = src.reshape(NC * NT, PAIRS, 2, R)
    dst_q = dst.reshape(NC * NT, PAIRS, 2, R)

    h1, a_s, a_d = _tc1(x, W1, att_src1.reshape(1, H1), att_dst1.reshape(1, H1))
    asp = jnp.pad(a_s.reshape(-1), (0, NP - N))
    adp = jnp.pad(a_d.reshape(-1), (0, NP - N))
    macc1, den1 = _sc128(h1, asp, adp, src_q, dst_q)
    den1t = den1.reshape(NC * NT, NP).T
    h1p, as2, ad2 = _tc2(macc1[0], macc1[1], den1t,
                         b1.reshape(1, H1), W2,
                         att_src2.reshape(1, H2), att_dst2.reshape(1, H2))
    macc2, den2 = _sc128(h1p, as2.reshape(-1), ad2.reshape(-1), src_q, dst_q)
    den2t = den2.reshape(NC * NT, NP).T
    out = _tc3(macc2[0], macc2[1], den2t, W2, b2.reshape(1, H2))
    return out[:N]


# speedup vs baseline: 32.3568x; 1.1045x over previous
import functools

import jax
import jax.numpy as jnp
from jax import lax
from jax.experimental import pallas as pl
from jax.experimental.pallas import tpu as pltpu
from jax.experimental.pallas import tpu_sc as plsc

N = 10000
D_IN = 128
H1 = 128
H2 = 16
NEG = 0.2

NT = 16
NC = 2
NP = 10112
DUMMY = N
R = 48
PAIRS = 108
EPP = NC * NT * PAIRS * 2 * R
RPT = NP // NT


def _tc1_body(x_ref, w_ref, asw_ref, adw_ref, h_ref, as_ref, ad_ref):
    h = jnp.dot(x_ref[...], w_ref[...], preferred_element_type=jnp.float32)
    h_ref[...] = h
    as_ref[...] = jnp.sum(h * asw_ref[...], axis=1, keepdims=True)
    ad_ref[...] = jnp.sum(h * adw_ref[...], axis=1, keepdims=True)


def _tc1(x, W1, asw, adw):
    blk = 1000
    return pl.pallas_call(
        _tc1_body,
        grid=(N // blk,),
        in_specs=[
            pl.BlockSpec((blk, D_IN), lambda i: (i, 0)),
            pl.BlockSpec((D_IN, H1), lambda i: (0, 0)),
            pl.BlockSpec((1, H1), lambda i: (0, 0)),
            pl.BlockSpec((1, H1), lambda i: (0, 0)),
        ],
        out_specs=[
            pl.BlockSpec((blk, H1), lambda i: (i, 0)),
            pl.BlockSpec((blk, 1), lambda i: (i, 0)),
            pl.BlockSpec((blk, 1), lambda i: (i, 0)),
        ],
        out_shape=[
            jax.ShapeDtypeStruct((N, H1), jnp.float32),
            jax.ShapeDtypeStruct((N, 1), jnp.float32),
            jax.ShapeDtypeStruct((N, 1), jnp.float32),
        ],
    )(x, W1, asw, adw)


def _tc2_body(m0, m1, dt, b1r, w2, asw, adw, h1p_ref, as_ref, ad_ref):
    den = jnp.sum(dt[...], axis=1, keepdims=True) + 1e-16
    h1 = jnp.maximum((m0[...] + m1[...]) / den + b1r[...], 0.0)
    h1p_ref[...] = h1
    h2 = jnp.dot(h1, w2[...], preferred_element_type=jnp.float32)
    as_ref[...] = jnp.sum(h2 * asw[...], axis=1, keepdims=True)
    ad_ref[...] = jnp.sum(h2 * adw[...], axis=1, keepdims=True)


def _tc2(m0, m1, dt, b1r, W2, asw, adw):
    return pl.pallas_call(
        _tc2_body,
        grid=(1,),
        in_specs=[
            pl.BlockSpec((NP, H1), lambda i: (0, 0)),
            pl.BlockSpec((NP, H1), lambda i: (0, 0)),
            pl.BlockSpec((NP, NC * NT), lambda i: (0, 0)),
            pl.BlockSpec((1, H1), lambda i: (0, 0)),
            pl.BlockSpec((H1, H2), lambda i: (0, 0)),
            pl.BlockSpec((1, H2), lambda i: (0, 0)),
            pl.BlockSpec((1, H2), lambda i: (0, 0)),
        ],
        out_specs=[
            pl.BlockSpec((NP, H1), lambda i: (0, 0)),
            pl.BlockSpec((NP, 1), lambda i: (0, 0)),
            pl.BlockSpec((NP, 1), lambda i: (0, 0)),
        ],
        out_shape=[
            jax.ShapeDtypeStruct((NP, H1), jnp.float32),
            jax.ShapeDtypeStruct((NP, 1), jnp.float32),
            jax.ShapeDtypeStruct((NP, 1), jnp.float32),
        ],
    )(m0, m1, dt, b1r, W2, asw, adw)


def _tc3_body(m0, m1, dt, w2, b2r, out_ref):
    den = jnp.sum(dt[...], axis=1, keepdims=True) + 1e-16
    agg = (m0[...] + m1[...]) / den
    out_ref[...] = (
        jnp.dot(agg, w2[...], preferred_element_type=jnp.float32) + b2r[...])


def _tc3(m0, m1, dt, W2, b2r):
    return pl.pallas_call(
        _tc3_body,
        grid=(1,),
        in_specs=[
            pl.BlockSpec((NP, H1), lambda i: (0, 0)),
            pl.BlockSpec((NP, H1), lambda i: (0, 0)),
            pl.BlockSpec((NP, NC * NT), lambda i: (0, 0)),
            pl.BlockSpec((H1, H2), lambda i: (0, 0)),
            pl.BlockSpec((1, H2), lambda i: (0, 0)),
        ],
        out_specs=pl.BlockSpec((NP, H2), lambda i: (0, 0)),
        out_shape=jax.ShapeDtypeStruct((NP, H2), jnp.float32),
    )(m0, m1, dt, W2, b2r)


def _make_sc_layer(C):
    mesh = plsc.VectorSubcoreMesh(core_axis_name="c", subcore_axis_name="s",
                                  num_cores=NC, num_subcores=NT)

    @functools.partial(
        pl.kernel,
        out_type=(
            jax.ShapeDtypeStruct((NC, NP, C), jnp.float32),
            jax.ShapeDtypeStruct((NC * NT * NP,), jnp.float32),
        ),
        mesh=mesh,
        compiler_params=pltpu.CompilerParams(needs_layout_passes=False),
        scratch_types=[
            pltpu.VMEM((NP,), jnp.float32),
            pltpu.VMEM((NP,), jnp.float32),
            pltpu.VMEM((NP,), jnp.float32),
            pltpu.VMEM((R,), jnp.int32),
            pltpu.VMEM((R,), jnp.int32),
            pltpu.VMEM((R,), jnp.int32),
            pltpu.VMEM((R,), jnp.int32),
            pltpu.VMEM((R,), jnp.int32),
            pltpu.VMEM((R,), jnp.int32),
            pltpu.VMEM((R,), jnp.float32),
            pltpu.VMEM((R,), jnp.float32),
            pltpu.VMEM((R, C), jnp.float32),
            pltpu.VMEM((R, C), jnp.float32),
            pltpu.VMEM_SHARED((NP, C), jnp.float32),
            pltpu.SemaphoreType.DMA,
            pltpu.SemaphoreType.DMA,
            pltpu.SemaphoreType.DMA,
            pltpu.SemaphoreType.DMA,
            pltpu.SemaphoreType.DMA,
        ],
    )
    def sc_layer(h_hbm, asrc_hbm, adst_hbm, src_hbm, dst_hbm,
                 macc_hbm, den_hbm,
                 asrc_v, adst_v, den_v, srcb_a, srcb_b, dstb_a, dstb_b,
                 dsc_a, dsc_b, exb_a, exb_b, hb_a, hb_b, acc_sh,
                 sem_g, sem_g2, sem_i, sem_sa, sem_sb):
        cid = lax.axis_index("c")
        sid = lax.axis_index("s")
        wid = sid * NC + cid
        z = jnp.zeros((16,), jnp.float32)

        def zrow(r, _):
            for k in range(C // 16):
                hb_a[r, pl.ds(k * 16, 16)] = z
            return 0
        lax.fori_loop(0, R, zrow, 0)

        def zden(i, _):
            den_v[pl.ds(i * 16, 16)] = z
            return 0
        lax.fori_loop(0, NP // 16, zden, 0)
        base = sid * RPT
        for b in range(RPT // R):
            pltpu.sync_copy(hb_a, acc_sh.at[pl.ds(base + b * R, R)])
        rem = RPT % R
        if rem:
            pltpu.sync_copy(hb_a.at[pl.ds(0, rem)],
                            acc_sh.at[pl.ds(base + (RPT // R) * R, rem)])

        pltpu.sync_copy(asrc_hbm, asrc_v)
        pltpu.sync_copy(adst_hbm, adst_v)
        plsc.subcore_barrier()

        def logits(srcb, dstb, exb):
            for i in range(R // 16):
                sv = srcb[pl.ds(i * 16, 16)]
                dv = dstb[pl.ds(i * 16, 16)]
                a = (plsc.load_gather(asrc_v, [sv])
                     + plsc.load_gather(adst_v, [dv]))
                a = jnp.where(a >= 0, a, a * NEG)
                ex = jnp.exp(a)
                exb[pl.ds(i * 16, 16)] = ex
                plsc.addupdate_scatter(den_v, [dv], ex)

        def med(hb, exb):
            def body(i, _):
                ev = exb[pl.ds(i * 16, 16)]
                for l in range(16):
                    bex = jnp.full((16,), ev[l], jnp.float32)
                    jj = i * 16 + l
                    for k in range(C // 16):
                        hb[jj, pl.ds(k * 16, 16)] = (
                            hb[jj, pl.ds(k * 16, 16)] * bex)
                return 0
            lax.fori_loop(0, R // 16, body, 0)

        for d in [pltpu.async_copy(src_hbm.at[wid, 0, 0], srcb_a, sem_i),
                  pltpu.async_copy(dst_hbm.at[wid, 0, 0], dstb_a, sem_i),
                  pltpu.async_copy(src_hbm.at[wid, 0, 1], srcb_b, sem_i),
                  pltpu.async_copy(dst_hbm.at[wid, 0, 1], dstb_b, sem_i)]:
            d.wait()

        def icopy(srcb, dstb):
            for i in range(R // 16):
                dstb[pl.ds(i * 16, 16)] = srcb[pl.ds(i * 16, 16)]

        for i in range(R // 16):
            dsc_a[pl.ds(i * 16, 16)] = jnp.full((16,), DUMMY, jnp.int32)
            dsc_b[pl.ds(i * 16, 16)] = jnp.full((16,), DUMMY, jnp.int32)
        pltpu.async_copy(hb_a, acc_sh.at[dsc_a], sem_sa, add=True)
        pltpu.async_copy(hb_a, acc_sh.at[dsc_b], sem_sb, add=True)

        def pair(k, _):
            kn = jnp.minimum(k + 1, PAIRS - 1)
            pltpu.make_async_copy(h_hbm.at[pl.ds(0, R)], hb_a, sem_sa).wait()
            ga = pltpu.async_copy(h_hbm.at[srcb_a], hb_a, sem_g)
            pltpu.make_async_copy(h_hbm.at[pl.ds(0, R)], hb_b, sem_sb).wait()
            gb = pltpu.async_copy(h_hbm.at[srcb_b], hb_b, sem_g2)
            logits(srcb_a, dstb_a, exb_a)
            icopy(dstb_a, dsc_a)
            ja = pltpu.async_copy(dst_hbm.at[wid, kn, 0], dstb_a, sem_i)
            ga.wait()
            ia = pltpu.async_copy(src_hbm.at[wid, kn, 0], srcb_a, sem_i)
            med(hb_a, exb_a)
            pltpu.async_copy(hb_a, acc_sh.at[dsc_a], sem_sa, add=True)
            logits(srcb_b, dstb_b, exb_b)
            icopy(dstb_b, dsc_b)
            jb = pltpu.async_copy(dst_hbm.at[wid, kn, 1], dstb_b, sem_i)
            gb.wait()
            ib = pltpu.async_copy(src_hbm.at[wid, kn, 1], srcb_b, sem_i)
            med(hb_b, exb_b)
            pltpu.async_copy(hb_b, acc_sh.at[dsc_b], sem_sb, add=True)
            ia.wait()
            ib.wait()
            ja.wait()
            jb.wait()
            return 0
        lax.fori_loop(0, PAIRS, pair, 0)
        pltpu.make_async_copy(h_hbm.at[pl.ds(0, R)], hb_a, sem_sa).wait()
        pltpu.make_async_copy(h_hbm.at[pl.ds(0, R)], hb_b, sem_sb).wait()

        plsc.subcore_barrier()
        pltpu.sync_copy(acc_sh.at[pl.ds(base, RPT)],
                        macc_hbm.at[cid, pl.ds(base, RPT)])
        pltpu.sync_copy(den_v, den_hbm.at[pl.ds((cid * NT + sid) * NP, NP)])

    return sc_layer


_sc128 = _make_sc_layer(H1)


def kernel(x, train_pos_edge_index, edge_weight, W1, att_src1, att_dst1, b1,
           W2, att_src2, att_dst2, b2):
    del edge_weight
    ei = train_pos_edge_index.astype(jnp.int32)
    loop = jnp.arange(N, dtype=jnp.int32)
    pad = EPP - (ei.shape[1] + N)
    src = jnp.concatenate([ei[0], loop, jnp.zeros((pad,), jnp.int32)])
    dst = jnp.concatenate([ei[1], loop, jnp.full((pad,), DUMMY, jnp.int32)])
    src_q = src.reshape(NC * NT, PAIRS, 2, R)
    dst_q = dst.reshape(NC * NT, PAIRS, 2, R)

    h1, a_s, a_d = _tc1(x, W1, att_src1.reshape(1, H1), att_dst1.reshape(1, H1))
    asp = jnp.pad(a_s.reshape(-1), (0, NP - N))
    adp = jnp.pad(a_d.reshape(-1), (0, NP - N))
    macc1, den1 = _sc128(h1, asp, adp, src_q, dst_q)
    den1t = den1.reshape(NC * NT, NP).T
    h1p, as2, ad2 = _tc2(macc1[0], macc1[1], den1t,
                         b1.reshape(1, H1), W2,
                         att_src2.reshape(1, H2), att_dst2.reshape(1, H2))
    macc2, den2 = _sc128(h1p, as2.reshape(-1), ad2.reshape(-1), src_q, dst_q)
    den2t = den2.reshape(NC * NT, NP).T
    out = _tc3(macc2[0], macc2[1], den2t, W2, b2.reshape(1, H2))
    return out[:N]
